# Initial kernel scaffold; baseline (speedup 1.0000x reference)
import functools

import jax
import jax.numpy as jnp
from jax import lax
from jax.experimental import pallas as pl
from jax.experimental.pallas import tpu as pltpu
from jax.experimental.pallas import tpu_sc as plsc

_B, _N, _M, _HA, _HB = 10, 1000, 32, 128, 16
_ROWS = _B * _N * _M
_NODES = _B * _N
_H2 = 2 * _HA

_NW = 32
_PER_W = _ROWS // _NW
_CHUNK = 80
_NCHUNK = _PER_W // _CHUNK


def _sc_gather(table, idx2d):
    mesh = plsc.VectorSubcoreMesh(core_axis_name="c", subcore_axis_name="s")

    @functools.partial(
        pl.kernel,
        out_type=jax.ShapeDtypeStruct((_ROWS, _HA), jnp.float32),
        mesh=mesh,
        scratch_types=[
            pltpu.VMEM((_PER_W,), jnp.int32),
            pltpu.VMEM((_CHUNK, _HA), jnp.float32),
            pltpu.SemaphoreType.DMA,
        ],
    )
    def k(table_hbm, idx_hbm, out_hbm, idx_v, rows_v, sem):
        wid = lax.axis_index("s") * 2 + lax.axis_index("c")
        base = wid * _PER_W
        pltpu.sync_copy(idx_hbm.at[wid], idx_v)

        def body(c, carry):
            off = c * _CHUNK
            pltpu.async_copy(
                table_hbm.at[idx_v.at[pl.ds(off, _CHUNK)]], rows_v, sem
            ).wait()
            pltpu.sync_copy(rows_v, out_hbm.at[pl.ds(base + off, _CHUNK)])
            return carry

        lax.fori_loop(0, _NCHUNK, body, 0)

    return k(table, idx2d)


_TN = 200
_TT = _NODES // _TN
_RT = _TN * _M
_TPB = _N // _TN


def _tc_main_body(g_ref, nb_ref, at_ref, mk_ref, w1_ref, w2_ref, w3_ref,
                  fcb_ref, bnhg_ref, bnhb_ref,
                  ns_ref, t1_ref, t2_ref,
                  s1, s2, a1, a2):
    p = pl.program_id(0)
    t = pl.program_id(1)

    @pl.when(jnp.logical_and(p == 0, t == 0))
    def _():
        s1[...] = jnp.zeros_like(s1)
        s2[...] = jnp.zeros_like(s2)

    @pl.when(jnp.logical_and(p == 1, t == 0))
    def _():
        a1[...] = jnp.zeros_like(a1)
        a2[...] = jnp.zeros_like(a2)

    g = g_ref[...] * mk_ref[...]
    nb = nb_ref[...].reshape(_RT, _HB)
    y = jnp.dot(g.astype(jnp.bfloat16), w2_ref[...],
                preferred_element_type=jnp.float32)
    y = y + jnp.dot(nb.astype(jnp.bfloat16), w3_ref[...],
                    preferred_element_type=jnp.float32)
    p1 = jnp.dot(at_ref[...].astype(jnp.bfloat16), w1_ref[...],
                 preferred_element_type=jnp.float32) + fcb_ref[...]
    y = y + jnp.broadcast_to(
        p1.reshape(_TN, 1, _H2), (_TN, _M, _H2)).reshape(_RT, _H2)

    @pl.when(p == 0)
    def _():
        s1[...] += jnp.sum(y, axis=0, keepdims=True)
        s2[...] += jnp.sum(y * y, axis=0, keepdims=True)

    @pl.when(p == 1)
    def _():
        mu = s1[...] * (1.0 / _ROWS)
        var = s2[...] * (1.0 / _ROWS) - mu * mu
        inv = lax.rsqrt(var + 1e-5)
        sc = bnhg_ref[...] * inv
        sh = bnhb_ref[...] - mu * sc
        yn = y * sc + sh
        f = jax.nn.sigmoid(yn[:, :_HA])
        c = jnp.maximum(yn[:, _HA:], 0.0)
        s = (f * c).reshape(_TN, _M, _HA).sum(axis=1)
        ns_ref[...] = s
        a1[...] += jnp.sum(s, axis=0, keepdims=True)
        a2[...] += jnp.sum(s * s, axis=0, keepdims=True)

        @pl.when(t == _TT - 1)
        def _():
            t1_ref[...] = a1[...]
            t2_ref[...] = a2[...]


def _tc_main(g, nbr_emb, atom2, mask3, w1t, w2t, w3t, fcb, bnhg, bnhb):
    return pl.pallas_call(
        _tc_main_body,
        grid=(2, _TT),
        in_specs=[
            pl.BlockSpec((_RT, _HA), lambda p, t: (t, 0)),
            pl.BlockSpec((1, _TN, _M, _HB),
                         lambda p, t: (t // _TPB, t % _TPB, 0, 0)),
            pl.BlockSpec((_TN, _HA), lambda p, t: (t, 0)),
            pl.BlockSpec((_RT, 1), lambda p, t: (t, 0)),
            pl.BlockSpec((_HA, _H2), lambda p, t: (0, 0)),
            pl.BlockSpec((_HA, _H2), lambda p, t: (0, 0)),
            pl.BlockSpec((_HB, _H2), lambda p, t: (0, 0)),
            pl.BlockSpec((1, _H2), lambda p, t: (0, 0)),
            pl.BlockSpec((1, _H2), lambda p, t: (0, 0)),
            pl.BlockSpec((1, _H2), lambda p, t: (0, 0)),
        ],
        out_specs=[
            pl.BlockSpec((_TN, _HA), lambda p, t: (t, 0)),
            pl.BlockSpec((1, _HA), lambda p, t: (0, 0)),
            pl.BlockSpec((1, _HA), lambda p, t: (0, 0)),
        ],
        out_shape=[
            jax.ShapeDtypeStruct((_NODES, _HA), jnp.float32),
            jax.ShapeDtypeStruct((1, _HA), jnp.float32),
            jax.ShapeDtypeStruct((1, _HA), jnp.float32),
        ],
        scratch_shapes=[
            pltpu.VMEM((1, _H2), jnp.float32),
            pltpu.VMEM((1, _H2), jnp.float32),
            pltpu.VMEM((1, _HA), jnp.float32),
            pltpu.VMEM((1, _HA), jnp.float32),
        ],
    )(g, nbr_emb, atom2, mask3, w1t, w2t, w3t, fcb, bnhg, bnhb)


def _tc_final_body(ns_ref, t1_ref, t2_ref, at_ref, bnog_ref, bnob_ref, o_ref):
    mu = t1_ref[...] * (1.0 / _NODES)
    var = t2_ref[...] * (1.0 / _NODES) - mu * mu
    inv = lax.rsqrt(var + 1e-5)
    sc = bnog_ref[...] * inv
    sh = bnob_ref[...] - mu * sc
    o_ref[...] = jnp.maximum(at_ref[...] + ns_ref[...] * sc + sh, 0.0)


def _tc_final(ns, t1, t2, atom2, bnog, bnob):
    return pl.pallas_call(
        _tc_final_body,
        out_shape=jax.ShapeDtypeStruct((_NODES, _HA), jnp.float32),
    )(ns, t1, t2, atom2, bnog, bnob)


def kernel(atom_emb, nbr_emb, atom_mask, fc_W, fc_b, bnh_g, bnh_b, bno_g,
           bno_b, nbr_adj_list):
    table = atom_emb.reshape(_NODES, _HA)
    flat_idx = (
        nbr_adj_list
        + (jnp.arange(_B, dtype=jnp.int32) * _N)[:, None, None]
    ).reshape(_NW, _PER_W)

    g = _sc_gather(table, flat_idx)

    w1t = fc_W[:, :_HA].T.astype(jnp.bfloat16)
    w2t = fc_W[:, _HA:2 * _HA].T.astype(jnp.bfloat16)
    w3t = fc_W[:, 2 * _HA:].T.astype(jnp.bfloat16)

    ns, t1, t2 = _tc_main(
        g, nbr_emb, table, atom_mask.reshape(_ROWS, 1),
        w1t, w2t, w3t,
        fc_b.reshape(1, _H2), bnh_g.reshape(1, _H2), bnh_b.reshape(1, _H2),
    )
    out = _tc_final(ns, t1, t2, table,
                    bno_g.reshape(1, _HA), bno_b.reshape(1, _HA))
    return out.reshape(_B, _N, _HA)


# speedup vs baseline: 6.1013x; 6.1013x over previous
import functools

import jax
import jax.numpy as jnp
from jax import lax
from jax.experimental import pallas as pl
from jax.experimental.pallas import tpu as pltpu
from jax.experimental.pallas import tpu_sc as plsc

_B, _N, _M, _HA, _HB = 10, 1000, 32, 128, 16
_ROWS = _B * _N * _M
_NODES = _B * _N
_H2 = 2 * _HA

_NW = 32
_PER_W = _ROWS // _NW
_CHUNK = 80
_NCHUNK = _PER_W // _CHUNK


def _sc_gather(table, idx2d):
    mesh = plsc.VectorSubcoreMesh(core_axis_name="c", subcore_axis_name="s")

    @functools.partial(
        pl.kernel,
        out_type=jax.ShapeDtypeStruct((_ROWS, _HA), jnp.float32),
        mesh=mesh,
        scratch_types=[
            pltpu.VMEM((_PER_W,), jnp.int32),
            pltpu.VMEM((_CHUNK, _HA), jnp.float32),
            pltpu.SemaphoreType.DMA,
        ],
    )
    def k(table_hbm, idx_hbm, out_hbm, idx_v, rows_v, sem):
        wid = lax.axis_index("s") * 2 + lax.axis_index("c")
        base = wid * _PER_W
        pltpu.sync_copy(idx_hbm.at[wid], idx_v)

        def body(c, carry):
            off = c * _CHUNK
            pltpu.async_copy(
                table_hbm.at[idx_v.at[pl.ds(off, _CHUNK)]], rows_v, sem
            ).wait()
            pltpu.sync_copy(rows_v, out_hbm.at[pl.ds(base + off, _CHUNK)])
            return carry

        lax.fori_loop(0, _NCHUNK, body, 0)

    return k(table, idx2d)


_TN = 200
_TT = _NODES // _TN
_RT = _TN * _M
_TPB = _N // _TN


def _tc_main_body(g_ref, nb_ref, at_ref, mk_ref, w1_ref, w2_ref, w3_ref,
                  fcb_ref, bnhg_ref, bnhb_ref,
                  ns_ref, t1_ref, t2_ref,
                  s1, s2, a1, a2):
    p = pl.program_id(0)
    t = pl.program_id(1)

    @pl.when(jnp.logical_and(p == 0, t == 0))
    def _():
        s1[...] = jnp.zeros_like(s1)
        s2[...] = jnp.zeros_like(s2)

    @pl.when(jnp.logical_and(p == 1, t == 0))
    def _():
        a1[...] = jnp.zeros_like(a1)
        a2[...] = jnp.zeros_like(a2)

    g = g_ref[...] * mk_ref[...]
    nb = nb_ref[...].reshape(_RT, _HB)
    y = jnp.dot(g.astype(jnp.bfloat16), w2_ref[...],
                preferred_element_type=jnp.float32)
    y = y + jnp.dot(nb.astype(jnp.bfloat16), w3_ref[...],
                    preferred_element_type=jnp.float32)
    p1 = jnp.dot(at_ref[...].astype(jnp.bfloat16), w1_ref[...],
                 preferred_element_type=jnp.float32) + fcb_ref[...]
    y = y + jnp.broadcast_to(
        p1.reshape(_TN, 1, _H2), (_TN, _M, _H2)).reshape(_RT, _H2)

    @pl.when(p == 0)
    def _():
        s1[...] += jnp.sum(y, axis=0, keepdims=True)
        s2[...] += jnp.sum(y * y, axis=0, keepdims=True)

    @pl.when(p == 1)
    def _():
        mu = s1[...] * (1.0 / _ROWS)
        var = s2[...] * (1.0 / _ROWS) - mu * mu
        inv = lax.rsqrt(var + 1e-5)
        sc = bnhg_ref[...] * inv
        sh = bnhb_ref[...] - mu * sc
        yn = y * sc + sh
        f = jax.nn.sigmoid(yn[:, :_HA])
        c = jnp.maximum(yn[:, _HA:], 0.0)
        s = (f * c).reshape(_TN, _M, _HA).sum(axis=1)
        ns_ref[...] = s
        a1[...] += jnp.sum(s, axis=0, keepdims=True)
        a2[...] += jnp.sum(s * s, axis=0, keepdims=True)

        @pl.when(t == _TT - 1)
        def _():
            t1_ref[...] = a1[...]
            t2_ref[...] = a2[...]


def _tc_main(g, nbr_emb, atom2, mask3, w1t, w2t, w3t, fcb, bnhg, bnhb):
    return pl.pallas_call(
        _tc_main_body,
        grid=(2, _TT),
        in_specs=[
            pl.BlockSpec((_RT, _HA), lambda p, t: (t, 0)),
            pl.BlockSpec((1, _TN, _M, _HB),
                         lambda p, t: (t // _TPB, t % _TPB, 0, 0)),
            pl.BlockSpec((_TN, _HA), lambda p, t: (t, 0)),
            pl.BlockSpec((_RT, 1), lambda p, t: (t, 0)),
            pl.BlockSpec((_HA, _H2), lambda p, t: (0, 0)),
            pl.BlockSpec((_HA, _H2), lambda p, t: (0, 0)),
            pl.BlockSpec((_HB, _H2), lambda p, t: (0, 0)),
            pl.BlockSpec((1, _H2), lambda p, t: (0, 0)),
            pl.BlockSpec((1, _H2), lambda p, t: (0, 0)),
            pl.BlockSpec((1, _H2), lambda p, t: (0, 0)),
        ],
        out_specs=[
            pl.BlockSpec((_TN, _HA), lambda p, t: (p * t, 0)),
            pl.BlockSpec((1, _HA), lambda p, t: (0, 0)),
            pl.BlockSpec((1, _HA), lambda p, t: (0, 0)),
        ],
        out_shape=[
            jax.ShapeDtypeStruct((_NODES, _HA), jnp.float32),
            jax.ShapeDtypeStruct((1, _HA), jnp.float32),
            jax.ShapeDtypeStruct((1, _HA), jnp.float32),
        ],
        scratch_shapes=[
            pltpu.VMEM((1, _H2), jnp.float32),
            pltpu.VMEM((1, _H2), jnp.float32),
            pltpu.VMEM((1, _HA), jnp.float32),
            pltpu.VMEM((1, _HA), jnp.float32),
        ],
    )(g, nbr_emb, atom2, mask3, w1t, w2t, w3t, fcb, bnhg, bnhb)


def _tc_final_body(ns_ref, t1_ref, t2_ref, at_ref, bnog_ref, bnob_ref, o_ref):
    mu = t1_ref[...] * (1.0 / _NODES)
    var = t2_ref[...] * (1.0 / _NODES) - mu * mu
    inv = lax.rsqrt(var + 1e-5)
    sc = bnog_ref[...] * inv
    sh = bnob_ref[...] - mu * sc
    o_ref[...] = jnp.maximum(at_ref[...] + ns_ref[...] * sc + sh, 0.0)


def _tc_final(ns, t1, t2, atom2, bnog, bnob):
    return pl.pallas_call(
        _tc_final_body,
        out_shape=jax.ShapeDtypeStruct((_NODES, _HA), jnp.float32),
    )(ns, t1, t2, atom2, bnog, bnob)


def kernel(atom_emb, nbr_emb, atom_mask, fc_W, fc_b, bnh_g, bnh_b, bno_g,
           bno_b, nbr_adj_list):
    table = atom_emb.reshape(_NODES, _HA)
    flat_idx = (
        nbr_adj_list
        + (jnp.arange(_B, dtype=jnp.int32) * _N)[:, None, None]
    ).reshape(_NW, _PER_W)

    g = _sc_gather(table, flat_idx)

    w1t = fc_W[:, :_HA].T.astype(jnp.bfloat16)
    w2t = fc_W[:, _HA:2 * _HA].T.astype(jnp.bfloat16)
    w3t = fc_W[:, 2 * _HA:].T.astype(jnp.bfloat16)

    ns, t1, t2 = _tc_main(
        g, nbr_emb, table, atom_mask.reshape(_ROWS, 1),
        w1t, w2t, w3t,
        fc_b.reshape(1, _H2), bnh_g.reshape(1, _H2), bnh_b.reshape(1, _H2),
    )
    out = _tc_final(ns, t1, t2, table,
                    bno_g.reshape(1, _HA), bno_b.reshape(1, _HA))
    return out.reshape(_B, _N, _HA)
